# Initial kernel scaffold; baseline (speedup 1.0000x reference)
import jax
import jax.numpy as jnp
from jax.experimental import pallas as pl


def kernel(indices, weight):
    raise NotImplementedError("write your pallas kernel here")


# speedup vs baseline: 1.5377x; 1.5377x over previous
import functools

import jax
import jax.numpy as jnp
from jax import lax
from jax.experimental import pallas as pl
from jax.experimental.pallas import tpu as pltpu
from jax.experimental.pallas import tpu_sc as plsc

_NUM_CLASSES = 1000000
_D = 32

_NC = 2
_NS = 16
_NW = _NC * _NS

_IDX_PER_GATHER = 128
_GATHERS_PER_CHUNK = 8
_CHUNK = _IDX_PER_GATHER * _GATHERS_PER_CHUNK


def _make_gather(batch):
    assert batch % (_NW * _CHUNK) == 0
    rows_per_w = batch // _NW
    n_chunks = rows_per_w // _CHUNK
    idx_rows_per_w = rows_per_w // _IDX_PER_GATHER

    mesh = plsc.VectorSubcoreMesh(core_axis_name="c", subcore_axis_name="s")

    @functools.partial(
        pl.kernel,
        out_type=jax.ShapeDtypeStruct((batch, _D), jnp.float32),
        mesh=mesh,
        compiler_params=pltpu.CompilerParams(use_tc_tiling_on_sc=False),
        scratch_types=[
            pltpu.VMEM((_GATHERS_PER_CHUNK, _IDX_PER_GATHER), jnp.int32),
            pltpu.VMEM((_CHUNK, _D), jnp.float32),
            pltpu.SemaphoreType.DMA,
        ],
    )
    def gather_kernel(table_hbm, idx_hbm, out_hbm, idx_v, rows_v, sem):
        wid = lax.axis_index("s") * _NC + lax.axis_index("c")
        idx_row0 = wid * idx_rows_per_w
        out_row0 = wid * rows_per_w

        def body(c, carry):
            pltpu.sync_copy(
                idx_hbm.at[pl.ds(idx_row0 + c * _GATHERS_PER_CHUNK,
                                 _GATHERS_PER_CHUNK)],
                idx_v,
            )
            copies = []
            for j in range(_GATHERS_PER_CHUNK):
                copies.append(
                    pltpu.async_copy(
                        table_hbm.at[idx_v.at[j]],
                        rows_v.at[pl.ds(j * _IDX_PER_GATHER, _IDX_PER_GATHER)],
                        sem,
                    ))
            for cp in copies:
                cp.wait()
            pltpu.sync_copy(rows_v, out_hbm.at[pl.ds(out_row0 + c * _CHUNK,
                                                     _CHUNK)])
            return carry

        lax.fori_loop(0, n_chunks, body, 0)

    return gather_kernel


def kernel(indices, weight):
    batch = indices.shape[0] * indices.shape[1]
    idx2d = indices.reshape(batch // _IDX_PER_GATHER, _IDX_PER_GATHER)
    out = _make_gather(batch)(weight, idx2d)
    return out.reshape(indices.shape + (_D,))


# speedup vs baseline: 1.5673x; 1.0192x over previous
import functools

import jax
import jax.numpy as jnp
from jax import lax
from jax.experimental import pallas as pl
from jax.experimental.pallas import tpu as pltpu
from jax.experimental.pallas import tpu_sc as plsc

_NUM_CLASSES = 1000000
_D = 32

_NC = 2
_NS = 16
_NW = _NC * _NS

_IDX_PER_GATHER = 128
_GATHERS_PER_CHUNK = 10
_CHUNK = _IDX_PER_GATHER * _GATHERS_PER_CHUNK


def _make_gather(batch):
    assert batch % (_NW * 2 * _CHUNK) == 0
    rows_per_w = batch // _NW
    n_chunks = rows_per_w // _CHUNK
    n_pairs = n_chunks // 2
    idx_rows_per_w = rows_per_w // _IDX_PER_GATHER

    mesh = plsc.VectorSubcoreMesh(core_axis_name="c", subcore_axis_name="s")

    @functools.partial(
        pl.kernel,
        out_type=jax.ShapeDtypeStruct((batch, _D), jnp.float32),
        mesh=mesh,
        compiler_params=pltpu.CompilerParams(use_tc_tiling_on_sc=False),
        scratch_types=[
            pltpu.VMEM((_GATHERS_PER_CHUNK, _IDX_PER_GATHER), jnp.int32),
            pltpu.VMEM((_GATHERS_PER_CHUNK, _IDX_PER_GATHER), jnp.int32),
            pltpu.VMEM((_CHUNK, _D), jnp.float32),
            pltpu.VMEM((_CHUNK, _D), jnp.float32),
            pltpu.SemaphoreType.DMA,
            pltpu.SemaphoreType.DMA,
            pltpu.SemaphoreType.DMA,
        ],
    )
    def gather_kernel(table_hbm, idx_hbm, out_hbm,
                      idx0, idx1, rows0, rows1, gsem, osem0, osem1):
        wid = lax.axis_index("s") * _NC + lax.axis_index("c")
        idx_row0 = wid * idx_rows_per_w
        out_row0 = wid * rows_per_w
        bufs = ((idx0, rows0, osem0), (idx1, rows1, osem1))

        def idx_copy(idx_v, c):
            pltpu.sync_copy(
                idx_hbm.at[pl.ds(idx_row0 + c * _GATHERS_PER_CHUNK,
                                 _GATHERS_PER_CHUNK)],
                idx_v,
            )

        def run_gathers(idx_v, rows_v):
            cps = []
            for j in range(_GATHERS_PER_CHUNK):
                cps.append(
                    pltpu.async_copy(
                        table_hbm.at[idx_v.at[j]],
                        rows_v.at[pl.ds(j * _IDX_PER_GATHER, _IDX_PER_GATHER)],
                        gsem,
                    ))
            for cp in cps:
                cp.wait()

        def out_slice(c):
            return out_hbm.at[pl.ds(out_row0 + c * _CHUNK, _CHUNK)]

        for b in range(2):
            idx_v, rows_v, osem = bufs[b]
            idx_copy(idx_v, b)
            run_gathers(idx_v, rows_v)
            pltpu.async_copy(rows_v, out_slice(b), osem)

        def pair(g, carry):
            for b in range(2):
                idx_v, rows_v, osem = bufs[b]
                c = 2 * g + b
                idx_copy(idx_v, c)
                pltpu.make_async_copy(rows_v, out_slice(c - 2), osem).wait()
                run_gathers(idx_v, rows_v)
                pltpu.async_copy(rows_v, out_slice(c), osem)
            return carry

        lax.fori_loop(1, n_pairs, pair, 0)

        for b in range(2):
            idx_v, rows_v, osem = bufs[b]
            pltpu.make_async_copy(rows_v, out_slice(n_chunks - 2 + b),
                                  osem).wait()

    return gather_kernel


def kernel(indices, weight):
    batch = indices.shape[0] * indices.shape[1]
    idx2d = indices.reshape(batch // _IDX_PER_GATHER, _IDX_PER_GATHER)
    out = _make_gather(batch)(weight, idx2d)
    return out.reshape(indices.shape + (_D,))


# speedup vs baseline: 1.5710x; 1.0024x over previous
import functools

import jax
import jax.numpy as jnp
from jax import lax
from jax.experimental import pallas as pl
from jax.experimental.pallas import tpu as pltpu
from jax.experimental.pallas import tpu_sc as plsc

_NUM_CLASSES = 1000000
_D = 32

_NC = 2
_NS = 16
_NW = _NC * _NS

_NB = 8


def _make_gather(batch, seqlen):
    assert batch % (_NW * 2 * _NB) == 0
    rows_per_w = batch // _NW
    n_chunks = rows_per_w // _NB
    n_pairs = n_chunks // 2

    mesh = plsc.VectorSubcoreMesh(core_axis_name="c", subcore_axis_name="s")

    @functools.partial(
        pl.kernel,
        out_type=jax.ShapeDtypeStruct((batch, seqlen, _D), jnp.float32),
        mesh=mesh,
        compiler_params=pltpu.CompilerParams(use_tc_tiling_on_sc=False),
        scratch_types=[
            pltpu.VMEM((_NB, seqlen), jnp.int32),
            pltpu.VMEM((_NB, seqlen), jnp.int32),
            pltpu.VMEM((_NB, seqlen, _D), jnp.float32),
            pltpu.VMEM((_NB, seqlen, _D), jnp.float32),
            pltpu.SemaphoreType.DMA,
            pltpu.SemaphoreType.DMA,
            pltpu.SemaphoreType.DMA,
        ],
    )
    def gather_kernel(table_hbm, idx_hbm, out_hbm,
                      idx0, idx1, rows0, rows1, gsem, osem0, osem1):
        wid = lax.axis_index("s") * _NC + lax.axis_index("c")
        row0 = wid * rows_per_w
        bufs = ((idx0, rows0, osem0), (idx1, rows1, osem1))

        def idx_copy(idx_v, c):
            pltpu.sync_copy(idx_hbm.at[pl.ds(row0 + c * _NB, _NB)], idx_v)

        def run_gathers(idx_v, rows_v):
            cps = []
            for i in range(_NB):
                cps.append(
                    pltpu.async_copy(
                        table_hbm.at[idx_v.at[i]], rows_v.at[i], gsem))
            for cp in cps:
                cp.wait()

        def out_slice(c):
            return out_hbm.at[pl.ds(row0 + c * _NB, _NB)]

        for b in range(2):
            idx_v, rows_v, osem = bufs[b]
            idx_copy(idx_v, b)
            run_gathers(idx_v, rows_v)
            pltpu.async_copy(rows_v, out_slice(b), osem)

        def pair(g, carry):
            for b in range(2):
                idx_v, rows_v, osem = bufs[b]
                c = 2 * g + b
                idx_copy(idx_v, c)
                pltpu.make_async_copy(rows_v, out_slice(c - 2), osem).wait()
                run_gathers(idx_v, rows_v)
                pltpu.async_copy(rows_v, out_slice(c), osem)
            return carry

        lax.fori_loop(1, n_pairs, pair, 0)

        for b in range(2):
            idx_v, rows_v, osem = bufs[b]
            pltpu.make_async_copy(rows_v, out_slice(n_chunks - 2 + b),
                                  osem).wait()

    return gather_kernel


def kernel(indices, weight):
    batch, seqlen = indices.shape
    return _make_gather(batch, seqlen)(weight, indices)
